# Initial kernel scaffold; baseline (speedup 1.0000x reference)
import jax
import jax.numpy as jnp
from jax.experimental import pallas as pl


def kernel(x, lang_map, max_lang_vocab_idx, weight):
    raise NotImplementedError("write your pallas kernel here")


# speedup vs baseline: 12.5252x; 12.5252x over previous
import functools

import jax
import jax.numpy as jnp
from jax import lax
from jax.experimental import pallas as pl
from jax.experimental.pallas import tpu as pltpu
from jax.experimental.pallas import tpu_sc as plsc

MAX_LANG_VOCAB_IDX = 100000
N_LANGS = 4096
EMBED_DIM = 64
BATCH = 16384
HIST = 200

N_TOKENS = BATCH * HIST
NW = 32
TOK_PER_W = N_TOKENS // NW
CHUNK = 128
N_CHUNKS = TOK_PER_W // CHUNK
LMAP_PAD = 100352


def _sc_kernel(x_hbm, lmap_hbm, cap_hbm, w_hbm, out_hbm,
               lmap_v, x_v, idx_v, rows_v, cap_v, sem):
    wid = lax.axis_index("s") * 2 + lax.axis_index("c")
    base_w = wid * TOK_PER_W

    pltpu.sync_copy(lmap_hbm, lmap_v)
    pltpu.sync_copy(cap_hbm, cap_v)
    cap = cap_v[...]

    def body(i, carry):
        base = base_w + i * CHUNK
        pltpu.sync_copy(x_hbm.at[pl.ds(base, CHUNK)], x_v)
        for j in range(CHUNK // 16):
            xv = x_v[pl.ds(j * 16, 16)]
            xc = jnp.minimum(xv, cap)
            y = plsc.load_gather(lmap_v, [xc])
            idx_v[pl.ds(j * 16, 16)] = y
        pltpu.async_copy(w_hbm.at[idx_v], rows_v, sem).wait()
        pltpu.sync_copy(rows_v, out_hbm.at[pl.ds(base, CHUNK)])
        return carry

    lax.fori_loop(0, N_CHUNKS, body, 0)


@jax.jit
def _run(x_flat, lmap_pad, cap, weight):
    mesh = plsc.VectorSubcoreMesh(core_axis_name="c", subcore_axis_name="s")
    f = functools.partial(
        pl.kernel,
        out_type=jax.ShapeDtypeStruct((N_TOKENS, EMBED_DIM), jnp.float32),
        mesh=mesh,
        compiler_params=pltpu.CompilerParams(
            needs_layout_passes=False, use_tc_tiling_on_sc=False
        ),
        scratch_types=[
            pltpu.VMEM((LMAP_PAD,), jnp.int32),
            pltpu.VMEM((CHUNK,), jnp.int32),
            pltpu.VMEM((CHUNK,), jnp.int32),
            pltpu.VMEM((CHUNK, EMBED_DIM), jnp.float32),
            pltpu.VMEM((16,), jnp.int32),
            pltpu.SemaphoreType.DMA,
        ],
    )(_sc_kernel)
    return f(x_flat, lmap_pad, cap, weight)


def kernel(x, lang_map, max_lang_vocab_idx, weight):
    x_flat = x.reshape(-1)
    lmap_pad = jnp.zeros((LMAP_PAD,), jnp.int32).at[: lang_map.shape[0]].set(lang_map)
    cap_vec = jnp.broadcast_to(max_lang_vocab_idx.astype(jnp.int32), (16,))
    out = _run(x_flat, lmap_pad, cap_vec, weight)
    return out.reshape(BATCH, HIST, EMBED_DIM)


# speedup vs baseline: 16.1313x; 1.2879x over previous
import functools

import jax
import jax.numpy as jnp
from jax import lax
from jax.experimental import pallas as pl
from jax.experimental.pallas import tpu as pltpu
from jax.experimental.pallas import tpu_sc as plsc

MAX_LANG_VOCAB_IDX = 100000
N_LANGS = 4096
EMBED_DIM = 64
BATCH = 16384
HIST = 200

N_TOKENS = BATCH * HIST
NW = 32
TOK_PER_W = N_TOKENS // NW
CHUNK = 128
NCHUNK = 8
STEP = CHUNK * NCHUNK
N_STEPS = TOK_PER_W // STEP
LMAP_PAD = 100352
LMAP_W = LMAP_PAD // 2


def _sc_kernel(x_hbm, lmap_hbm, cap_hbm, w_hbm, out_hbm,
               lmap_v, x_v, idx_v, rows_v, cap_v, gsems, wsem):
    wid = lax.axis_index("s") * 2 + lax.axis_index("c")
    base_w = wid * TOK_PER_W

    pltpu.sync_copy(lmap_hbm, lmap_v)
    pltpu.sync_copy(cap_hbm, cap_v)
    cap = cap_v[...]

    def body(s, carry):
        base = base_w + s * STEP
        pltpu.sync_copy(x_hbm.at[pl.ds(base, STEP)], x_v)
        for u in range(NCHUNK):
            for j in range(CHUNK // 16):
                xv = x_v[pl.ds(u * CHUNK + j * 16, 16)]
                xc = jnp.minimum(xv, cap)
                word = plsc.load_gather(
                    lmap_v, [lax.shift_right_logical(xc, 1)]
                )
                sh = jnp.left_shift(jnp.bitwise_and(xc, 1), 4)
                y = jnp.bitwise_and(
                    lax.shift_right_logical(word, sh), 0xFFFF
                )
                idx_v[u, pl.ds(j * 16, 16)] = y
        gathers = [
            pltpu.async_copy(
                w_hbm.at[idx_v.at[u]], rows_v.at[u], gsems[u]
            )
            for u in range(NCHUNK)
        ]
        for g in gathers:
            g.wait()
        writes = [
            pltpu.async_copy(
                rows_v.at[u],
                out_hbm.at[pl.ds(base + u * CHUNK, CHUNK)],
                wsem,
            )
            for u in range(NCHUNK)
        ]
        for w in writes:
            w.wait()
        return carry

    lax.fori_loop(0, N_STEPS, body, 0)


@jax.jit
def _run(x_flat, lmap_packed, cap, weight):
    mesh = plsc.VectorSubcoreMesh(core_axis_name="c", subcore_axis_name="s")
    f = functools.partial(
        pl.kernel,
        out_type=jax.ShapeDtypeStruct((N_TOKENS, EMBED_DIM), jnp.float32),
        mesh=mesh,
        compiler_params=pltpu.CompilerParams(
            needs_layout_passes=False, use_tc_tiling_on_sc=False
        ),
        scratch_types=[
            pltpu.VMEM((LMAP_W,), jnp.int32),
            pltpu.VMEM((STEP,), jnp.int32),
            pltpu.VMEM((NCHUNK, CHUNK), jnp.int32),
            pltpu.VMEM((NCHUNK, CHUNK, EMBED_DIM), jnp.float32),
            pltpu.VMEM((16,), jnp.int32),
            [pltpu.SemaphoreType.DMA] * NCHUNK,
            pltpu.SemaphoreType.DMA,
        ],
    )(_sc_kernel)
    return f(x_flat, lmap_packed, cap, weight)


def kernel(x, lang_map, max_lang_vocab_idx, weight):
    x_flat = x.reshape(-1)
    lmap_pad = jnp.zeros((LMAP_PAD,), jnp.int32).at[: lang_map.shape[0]].set(lang_map)
    lmap_packed = lmap_pad[0::2] | jnp.left_shift(lmap_pad[1::2], 16)
    cap_vec = jnp.broadcast_to(max_lang_vocab_idx.astype(jnp.int32), (16,))
    out = _run(x_flat, lmap_packed, cap_vec, weight)
    return out.reshape(BATCH, HIST, EMBED_DIM)


# speedup vs baseline: 16.1837x; 1.0032x over previous
import functools

import jax
import jax.numpy as jnp
from jax import lax
from jax.experimental import pallas as pl
from jax.experimental.pallas import tpu as pltpu
from jax.experimental.pallas import tpu_sc as plsc

MAX_LANG_VOCAB_IDX = 100000
N_LANGS = 4096
EMBED_DIM = 64
BATCH = 16384
HIST = 200

N_TOKENS = BATCH * HIST
NW = 32
TOK_PER_W = N_TOKENS // NW
CHUNK = 128
NGRP = 2
GCHUNK = 4
STEP = CHUNK * GCHUNK * NGRP
N_STEPS = TOK_PER_W // STEP
LMAP_PAD = 100352
LMAP_W = LMAP_PAD // 2


def _sc_kernel(x_hbm, lmap_hbm, cap_hbm, w_hbm, out_hbm,
               lmap_v, x_v, idx_v, rows_v, cap_v, xsem, gsems, wsems):
    wid = lax.axis_index("s") * 2 + lax.axis_index("c")
    base_w = wid * TOK_PER_W

    pltpu.sync_copy(lmap_hbm, lmap_v)
    pltpu.sync_copy(cap_hbm, cap_v)
    cap = cap_v[...]

    def x_block(s):
        base = jnp.minimum(base_w + s * STEP, N_TOKENS - STEP)
        return x_hbm.at[pl.ds(base, STEP)]

    def out_slice(base, p, u):
        off = base + (p * GCHUNK + u) * CHUNK
        return out_hbm.at[pl.ds(off, CHUNK)]

    pltpu.async_copy(x_block(0), x_v, xsem)

    def body(s, carry):
        base = base_w + s * STEP
        pltpu.make_async_copy(x_block(s), x_v, xsem).wait()

        gathers = []
        for p in range(NGRP):
            for u in range(GCHUNK):
                for j in range(CHUNK // 16):
                    t = (p * GCHUNK + u) * CHUNK + j * 16
                    xv = x_v[pl.ds(t, 16)]
                    xc = jnp.minimum(xv, cap)
                    word = plsc.load_gather(
                        lmap_v, [lax.shift_right_logical(xc, 1)]
                    )
                    sh = jnp.left_shift(jnp.bitwise_and(xc, 1), 4)
                    y = jnp.bitwise_and(
                        lax.shift_right_logical(word, sh), 0xFFFF
                    )
                    idx_v[p, u, pl.ds(j * 16, 16)] = y

            @pl.when(s > 0)
            def _():
                for u in range(GCHUNK):
                    pltpu.make_async_copy(
                        rows_v.at[p, u], out_slice(base, p, u), wsems[p]
                    ).wait()

            gathers.append([
                pltpu.async_copy(
                    w_hbm.at[idx_v.at[p, u]], rows_v.at[p, u], gsems[p]
                )
                for u in range(GCHUNK)
            ])

        pltpu.async_copy(x_block(s + 1), x_v, xsem)

        for p in range(NGRP):
            for g in gathers[p]:
                g.wait()
            for u in range(GCHUNK):
                pltpu.async_copy(
                    rows_v.at[p, u], out_slice(base, p, u), wsems[p]
                )
        return carry

    lax.fori_loop(0, N_STEPS, body, 0)

    last = base_w + (N_STEPS - 1) * STEP
    for p in range(NGRP):
        for u in range(GCHUNK):
            pltpu.make_async_copy(
                rows_v.at[p, u], out_slice(last, p, u), wsems[p]
            ).wait()
    pltpu.make_async_copy(x_block(N_STEPS), x_v, xsem).wait()


@jax.jit
def _run(x_flat, lmap_packed, cap, weight):
    mesh = plsc.VectorSubcoreMesh(core_axis_name="c", subcore_axis_name="s")
    f = functools.partial(
        pl.kernel,
        out_type=jax.ShapeDtypeStruct((N_TOKENS, EMBED_DIM), jnp.float32),
        mesh=mesh,
        compiler_params=pltpu.CompilerParams(
            needs_layout_passes=False, use_tc_tiling_on_sc=False
        ),
        scratch_types=[
            pltpu.VMEM((LMAP_W,), jnp.int32),
            pltpu.VMEM((STEP,), jnp.int32),
            pltpu.VMEM((NGRP, GCHUNK, CHUNK), jnp.int32),
            pltpu.VMEM((NGRP, GCHUNK, CHUNK, EMBED_DIM), jnp.float32),
            pltpu.VMEM((16,), jnp.int32),
            pltpu.SemaphoreType.DMA,
            [pltpu.SemaphoreType.DMA] * NGRP,
            [pltpu.SemaphoreType.DMA] * NGRP,
        ],
    )(_sc_kernel)
    return f(x_flat, lmap_packed, cap, weight)


def kernel(x, lang_map, max_lang_vocab_idx, weight):
    x_flat = x.reshape(-1)
    lmap_pad = jnp.zeros((LMAP_PAD,), jnp.int32).at[: lang_map.shape[0]].set(lang_map)
    lmap_packed = lmap_pad[0::2] | jnp.left_shift(lmap_pad[1::2], 16)
    cap_vec = jnp.broadcast_to(max_lang_vocab_idx.astype(jnp.int32), (16,))
    out = _run(x_flat, lmap_packed, cap_vec, weight)
    return out.reshape(BATCH, HIST, EMBED_DIM)


# speedup vs baseline: 21.5498x; 1.3316x over previous
import functools

import jax
import jax.numpy as jnp
from jax import lax
from jax.experimental import pallas as pl
from jax.experimental.pallas import tpu as pltpu
from jax.experimental.pallas import tpu_sc as plsc
from jax.experimental.layout import Format, Layout, with_layout_constraint

MAX_LANG_VOCAB_IDX = 100000
N_LANGS = 4096
EMBED_DIM = 64
BATCH = 16384
HIST = 200

N_TOKENS = BATCH * HIST
NW = 32
TOK_PER_W = N_TOKENS // NW
CHUNK = 128
NGRP = 2
GCHUNK = 4
STEP = CHUNK * GCHUNK * NGRP
N_STEPS = TOK_PER_W // STEP
LMAP_PAD = 100352
LMAP_W = LMAP_PAD // 2


def _sc_kernel(x_hbm, lmap_hbm, cap_hbm, w_hbm, out_hbm,
               lmap_v, x_v, idx_v, rows_v, cap_v, xsem, gsems, wsems):
    wid = lax.axis_index("s") * 2 + lax.axis_index("c")
    base_w = wid * TOK_PER_W

    pltpu.sync_copy(lmap_hbm, lmap_v)
    pltpu.sync_copy(cap_hbm, cap_v)
    cap = cap_v[...]

    def x_block(s):
        base = jnp.minimum(base_w + s * STEP, N_TOKENS - STEP)
        return x_hbm.at[pl.ds(base, STEP)]

    def out_slice(base, p, u):
        off = base + (p * GCHUNK + u) * CHUNK
        return out_hbm.at[pl.ds(off, CHUNK)]

    pltpu.async_copy(x_block(0), x_v, xsem)

    def body(s, carry):
        base = base_w + s * STEP
        pltpu.make_async_copy(x_block(s), x_v, xsem).wait()

        gathers = []
        for p in range(NGRP):
            for u in range(GCHUNK):
                for j in range(CHUNK // 16):
                    t = (p * GCHUNK + u) * CHUNK + j * 16
                    xv = x_v[pl.ds(t, 16)]
                    xc = jnp.minimum(xv, cap)
                    word = plsc.load_gather(
                        lmap_v, [lax.shift_right_logical(xc, 1)]
                    )
                    sh = jnp.left_shift(jnp.bitwise_and(xc, 1), 4)
                    y = jnp.bitwise_and(
                        lax.shift_right_logical(word, sh), 0xFFFF
                    )
                    idx_v[p, u, pl.ds(j * 16, 16)] = y

            @pl.when(s > 0)
            def _():
                for u in range(GCHUNK):
                    pltpu.make_async_copy(
                        rows_v.at[p, u], out_slice(base, p, u), wsems[p]
                    ).wait()

            gathers.append([
                pltpu.async_copy(
                    w_hbm.at[idx_v.at[p, u]], rows_v.at[p, u], gsems[p]
                )
                for u in range(GCHUNK)
            ])

        pltpu.async_copy(x_block(s + 1), x_v, xsem)

        for p in range(NGRP):
            for g in gathers[p]:
                g.wait()
            for u in range(GCHUNK):
                pltpu.async_copy(
                    rows_v.at[p, u], out_slice(base, p, u), wsems[p]
                )
        return carry

    lax.fori_loop(0, N_STEPS, body, 0)

    last = base_w + (N_STEPS - 1) * STEP
    for p in range(NGRP):
        for u in range(GCHUNK):
            pltpu.make_async_copy(
                rows_v.at[p, u], out_slice(last, p, u), wsems[p]
            ).wait()
    pltpu.make_async_copy(x_block(N_STEPS), x_v, xsem).wait()


@jax.jit
def _run(x_flat, lmap_packed, cap, weight):
    mesh = plsc.VectorSubcoreMesh(core_axis_name="c", subcore_axis_name="s")
    f = functools.partial(
        pl.kernel,
        out_type=jax.ShapeDtypeStruct((N_TOKENS, EMBED_DIM), jnp.float32),
        mesh=mesh,
        compiler_params=pltpu.CompilerParams(
            needs_layout_passes=False, use_tc_tiling_on_sc=False
        ),
        scratch_types=[
            pltpu.VMEM((LMAP_W,), jnp.int32),
            pltpu.VMEM((STEP,), jnp.int32),
            pltpu.VMEM((NGRP, GCHUNK, CHUNK), jnp.int32),
            pltpu.VMEM((NGRP, GCHUNK, CHUNK, EMBED_DIM), jnp.float32),
            pltpu.VMEM((16,), jnp.int32),
            pltpu.SemaphoreType.DMA,
            [pltpu.SemaphoreType.DMA] * NGRP,
            [pltpu.SemaphoreType.DMA] * NGRP,
        ],
    )(_sc_kernel)
    return f(x_flat, lmap_packed, cap, weight)


def kernel(x, lang_map, max_lang_vocab_idx, weight):
    x_flat = x.reshape(-1)
    lmap_pad = jnp.zeros((LMAP_PAD,), jnp.int32).at[: lang_map.shape[0]].set(lang_map)
    lmap_packed = lmap_pad[0::2] | jnp.left_shift(lmap_pad[1::2], 16)
    cap_vec = jnp.broadcast_to(max_lang_vocab_idx.astype(jnp.int32), (16,))
    out = _run(x_flat, lmap_packed, cap_vec, weight)
    out = out.reshape(BATCH, HIST, EMBED_DIM)
    return with_layout_constraint(out, Layout((0, 1, 2)))


# speedup vs baseline: 28.9068x; 1.3414x over previous
import functools

import jax
import jax.numpy as jnp
from jax import lax
from jax.experimental import pallas as pl
from jax.experimental.pallas import tpu as pltpu
from jax.experimental.pallas import tpu_sc as plsc

MAX_LANG_VOCAB_IDX = 100000
N_LANGS = 4096
EMBED_DIM = 64
BATCH = 16384
HIST = 200

N_TOKENS = BATCH * HIST
NW = 32
TOK_PER_W = N_TOKENS // NW
CHUNK = 128
NGRP = 2
GCHUNK = 4
STEP = CHUNK * GCHUNK * NGRP
N_STEPS = TOK_PER_W // STEP
LMAP_PAD = 100352
LMAP_W = LMAP_PAD // 2


def _sc_kernel(x_hbm, lmap_hbm, cap_hbm, w_hbm, out_hbm,
               lmap_v, x_v, idx_v, rows_v, cap_v, xsem, gsems, wsems):
    wid = lax.axis_index("s") * 2 + lax.axis_index("c")
    base_w = wid * TOK_PER_W

    pltpu.sync_copy(lmap_hbm, lmap_v)
    pltpu.sync_copy(cap_hbm, cap_v)
    cap = cap_v[...]

    def x_block(s):
        base = jnp.minimum(base_w + s * STEP, N_TOKENS - STEP)
        return x_hbm.at[pl.ds(base, STEP)]

    def out_slice(base, p, u):
        off = base + (p * GCHUNK + u) * CHUNK
        return out_hbm.at[pl.ds(off, CHUNK), pl.ds(0, EMBED_DIM)]

    pltpu.async_copy(x_block(0), x_v, xsem)

    def body(s, carry):
        base = base_w + s * STEP
        pltpu.make_async_copy(x_block(s), x_v, xsem).wait()

        gathers = []
        for p in range(NGRP):
            for u in range(GCHUNK):
                for j in range(CHUNK // 16):
                    t = (p * GCHUNK + u) * CHUNK + j * 16
                    xv = x_v[pl.ds(t, 16)]
                    xc = jnp.minimum(xv, cap)
                    word = plsc.load_gather(
                        lmap_v, [lax.shift_right_logical(xc, 1)]
                    )
                    sh = jnp.left_shift(jnp.bitwise_and(xc, 1), 4)
                    y = jnp.bitwise_and(
                        lax.shift_right_logical(word, sh), 0xFFFF
                    )
                    idx_v[p, u, pl.ds(j * 16, 16)] = y

            @pl.when(s > 0)
            def _():
                for u in range(GCHUNK):
                    pltpu.make_async_copy(
                        rows_v.at[p, u], out_slice(base, p, u), wsems[p]
                    ).wait()

            gathers.append([
                pltpu.async_copy(
                    w_hbm.at[idx_v.at[p, u]], rows_v.at[p, u], gsems[p]
                )
                for u in range(GCHUNK)
            ])

        pltpu.async_copy(x_block(s + 1), x_v, xsem)

        for p in range(NGRP):
            for g in gathers[p]:
                g.wait()
            for u in range(GCHUNK):
                pltpu.async_copy(
                    rows_v.at[p, u], out_slice(base, p, u), wsems[p]
                )
        return carry

    lax.fori_loop(0, N_STEPS, body, 0)

    last = base_w + (N_STEPS - 1) * STEP
    for p in range(NGRP):
        for u in range(GCHUNK):
            pltpu.make_async_copy(
                rows_v.at[p, u], out_slice(last, p, u), wsems[p]
            ).wait()
    pltpu.make_async_copy(x_block(N_STEPS), x_v, xsem).wait()


@jax.jit
def _run(x_flat, lmap_packed, cap, weight):
    mesh = plsc.VectorSubcoreMesh(core_axis_name="c", subcore_axis_name="s")
    f = functools.partial(
        pl.kernel,
        out_type=jax.ShapeDtypeStruct((N_TOKENS, 128), jnp.float32),
        mesh=mesh,
        compiler_params=pltpu.CompilerParams(
            needs_layout_passes=False, use_tc_tiling_on_sc=False
        ),
        scratch_types=[
            pltpu.VMEM((LMAP_W,), jnp.int32),
            pltpu.VMEM((STEP,), jnp.int32),
            pltpu.VMEM((NGRP, GCHUNK, CHUNK), jnp.int32),
            pltpu.VMEM((NGRP, GCHUNK, CHUNK, EMBED_DIM), jnp.float32),
            pltpu.VMEM((16,), jnp.int32),
            pltpu.SemaphoreType.DMA,
            [pltpu.SemaphoreType.DMA] * NGRP,
            [pltpu.SemaphoreType.DMA] * NGRP,
        ],
    )(_sc_kernel)
    return f(x_flat, lmap_packed, cap, weight)


def kernel(x, lang_map, max_lang_vocab_idx, weight):
    x_flat = x.reshape(-1)
    lmap_pad = jnp.zeros((LMAP_PAD,), jnp.int32).at[: lang_map.shape[0]].set(lang_map)
    lmap_packed = lmap_pad[0::2] | jnp.left_shift(lmap_pad[1::2], 16)
    cap_vec = jnp.broadcast_to(max_lang_vocab_idx.astype(jnp.int32), (16,))
    out = _run(x_flat, lmap_packed, cap_vec, weight)
    return out[:, :EMBED_DIM].reshape(BATCH, HIST, EMBED_DIM)
